# baseline (speedup 1.0000x reference)
import functools

import jax
import jax.numpy as jnp
from jax import lax
from jax.experimental import pallas as pl
from jax.experimental.pallas import tpu as pltpu
from jax.experimental.pallas import tpu_sc as plsc

D = 128
W16 = 16

NPAD = 10240
EPAD = 327680
NC = 2
NS = 16
NW = NC * NS
K = 80
NCHT = EPAD // K
NCHUNK = NCHT // NW
RPS = NPAD // NS
NBUF = 4

_MESH = dict(core_axis_name="c", subcore_axis_name="s", num_cores=NC,
             num_subcores=NS)


def _make_agg(d, sup):
    mesh = plsc.VectorSubcoreMesh(**_MESH)
    nsup = NCHUNK // sup

    @functools.partial(
        pl.kernel,
        out_type=jax.ShapeDtypeStruct((NC, NPAD, d), jnp.float32),
        mesh=mesh,
        compiler_params=pltpu.CompilerParams(use_tc_tiling_on_sc=False),
        scratch_types=[
            pltpu.VMEM((sup, K), jnp.int32),
            pltpu.VMEM((sup, K), jnp.int32),
        ] + [pltpu.VMEM((K, d), jnp.float32) for _ in range(NBUF)]
          + [pltpu.VMEM_SHARED((NPAD, d), jnp.float32)]
          + [pltpu.SemaphoreType.DMA for _ in range(2 * NBUF)],
    )
    def agg(tbl, srcp, dstp, out, src_a, dst_a, r0, r1, r2, r3, acc,
            g0, g1, g2, g3, s0, s1, s2, s3):
        bufs = (r0, r1, r2, r3)
        gsem = (g0, g1, g2, g3)
        ssem = (s0, s1, s2, s3)
        c = lax.axis_index("c")
        s = lax.axis_index("s")
        wid = s * NC + c
        cbase = wid * NCHUNK

        def zrow(r, _):
            def zcol(cc, _):
                r0[r, pl.ds(cc * 16, 16)] = jnp.zeros((16,), jnp.float32)
                return 0
            return lax.fori_loop(0, d // 16, zcol, 0)
        lax.fori_loop(0, K, zrow, 0)

        def zstripe(j, _):
            pltpu.sync_copy(r0, acc.at[pl.ds(s * RPS + j * K, K)])
            return 0
        lax.fori_loop(0, RPS // K, zstripe, 0)
        plsc.subcore_barrier()

        def start_g(i, b, sem):
            pltpu.async_copy(tbl.at[src_a.at[i]], b, sem)

        def start_s(i, b, sem):
            pltpu.async_copy(b, acc.at[dst_a.at[i]], sem, add=True)

        def wait_dma(b, sem):
            pltpu.make_async_copy(tbl.at[pl.ds(0, K)], b, sem).wait()

        def super_chunk(g, _):
            sbase = cbase + g * sup
            pltpu.sync_copy(srcp.at[pl.ds(sbase, sup)], src_a)
            pltpu.sync_copy(dstp.at[pl.ds(sbase, sup)], dst_a)
            for u in range(NBUF):
                start_g(u, bufs[u], gsem[u])

            def grp(j, _):
                for u in range(NBUF):
                    wait_dma(bufs[u], gsem[u])
                    start_s(NBUF * j + u, bufs[u], ssem[u])
                for u in range(NBUF):
                    wait_dma(bufs[u], ssem[u])
                    start_g(NBUF * (j + 1) + u, bufs[u], gsem[u])
                return 0
            lax.fori_loop(0, sup // NBUF - 1, grp, 0)

            tail = sup - NBUF
            for u in range(NBUF):
                wait_dma(bufs[u], gsem[u])
                start_s(tail + u, bufs[u], ssem[u])
            for u in range(NBUF):
                wait_dma(bufs[u], ssem[u])
            return 0
        lax.fori_loop(0, nsup, super_chunk, 0)

        plsc.subcore_barrier()
        pltpu.sync_copy(acc.at[pl.ds(s * RPS, RPS)],
                        out.at[c, pl.ds(s * RPS, RPS)])

    return agg


_agg128 = _make_agg(D, 16)
_agg16 = _make_agg(W16, NCHUNK)


def _make_deg():
    mesh = plsc.VectorSubcoreMesh(**_MESH)

    @functools.partial(
        pl.kernel,
        out_type=jax.ShapeDtypeStruct((NC, NPAD, W16), jnp.float32),
        mesh=mesh,
        compiler_params=pltpu.CompilerParams(use_tc_tiling_on_sc=False),
        scratch_types=[
            pltpu.VMEM((NCHUNK, K), jnp.int32),
            pltpu.VMEM((K, W16), jnp.float32),
            pltpu.VMEM((K, W16), jnp.float32),
            pltpu.VMEM_SHARED((NPAD, W16), jnp.float32),
            pltpu.SemaphoreType.DMA,
        ],
    )
    def deg(dstp, out, dst_a, ones_v, zb, acc, sem):
        c = lax.axis_index("c")
        s = lax.axis_index("s")
        wid = s * NC + c

        pltpu.sync_copy(dstp.at[pl.ds(wid * NCHUNK, NCHUNK)], dst_a)

        def fill(r, _):
            ones_v[r, pl.ds(0, 16)] = jnp.ones((16,), jnp.float32)
            zb[r, pl.ds(0, 16)] = jnp.zeros((16,), jnp.float32)
            return 0
        lax.fori_loop(0, K, fill, 0)

        def zstripe(j, _):
            pltpu.sync_copy(zb, acc.at[pl.ds(s * RPS + j * K, K)])
            return 0
        lax.fori_loop(0, RPS // K, zstripe, 0)
        plsc.subcore_barrier()

        def chunk(i, _):
            pltpu.sync_copy(ones_v, acc.at[dst_a.at[i]], add=True)
            return 0
        lax.fori_loop(0, NCHUNK, chunk, 0)
        plsc.subcore_barrier()

        pltpu.sync_copy(acc.at[pl.ds(s * RPS, RPS)],
                        out.at[c, pl.ds(s * RPS, RPS)])

    return deg


_deg = _make_deg()

BN = 256
GRID = NPAD // BN


def _dinv_of(degr):
    deg = (jnp.sum(degr[0], axis=1, keepdims=True)
           + jnp.sum(degr[1], axis=1, keepdims=True)) * (1.0 / W16) + 1.0
    return lax.rsqrt(deg)


def _tcb_body(xr, w1r, degr, hsr):
    dinv = _dinv_of(degr)
    hsr[...] = jnp.dot(xr[...], w1r[...],
                       preferred_element_type=jnp.float32) * dinv


_tcb = pl.pallas_call(
    _tcb_body,
    grid=(GRID,),
    in_specs=[
        pl.BlockSpec((BN, D), lambda i: (i, 0)),
        pl.BlockSpec((D, D), lambda i: (0, 0)),
        pl.BlockSpec((NC, BN, W16), lambda i: (0, i, 0)),
    ],
    out_specs=pl.BlockSpec((BN, D), lambda i: (i, 0)),
    out_shape=jax.ShapeDtypeStruct((NPAD, D), jnp.float32),
)


def _tcd_body(pr, hsr, degr, b1r, w2r, xr, wsr, br, h2sr, baser):
    dinv = _dinv_of(degr)
    h = jnp.maximum((pr[0] + pr[1] + hsr[...]) * dinv + b1r[...], 0.0)
    h2sr[...] = jnp.dot(h, w2r[...], preferred_element_type=jnp.float32) * dinv
    baser[...] = jnp.dot(xr[...], wsr[...],
                         preferred_element_type=jnp.float32) + br[...]


_tcd = pl.pallas_call(
    _tcd_body,
    grid=(GRID,),
    in_specs=[
        pl.BlockSpec((NC, BN, D), lambda i: (0, i, 0)),
        pl.BlockSpec((BN, D), lambda i: (i, 0)),
        pl.BlockSpec((NC, BN, W16), lambda i: (0, i, 0)),
        pl.BlockSpec((1, D), lambda i: (0, 0)),
        pl.BlockSpec((D, W16), lambda i: (0, 0)),
        pl.BlockSpec((BN, D), lambda i: (i, 0)),
        pl.BlockSpec((D, W16), lambda i: (0, 0)),
        pl.BlockSpec((1, W16), lambda i: (0, 0)),
    ],
    out_specs=[
        pl.BlockSpec((BN, W16), lambda i: (i, 0)),
        pl.BlockSpec((BN, W16), lambda i: (i, 0)),
    ],
    out_shape=[
        jax.ShapeDtypeStruct((NPAD, W16), jnp.float32),
        jax.ShapeDtypeStruct((NPAD, W16), jnp.float32),
    ],
)


def _tcf_body(qr, h2sr, baser, degr, outr):
    dinv = _dinv_of(degr)
    outr[...] = (qr[0] + qr[1] + h2sr[...]) * dinv + baser[...]


_tcf = pl.pallas_call(
    _tcf_body,
    grid=(GRID,),
    in_specs=[
        pl.BlockSpec((NC, BN, W16), lambda i: (0, i, 0)),
        pl.BlockSpec((BN, W16), lambda i: (i, 0)),
        pl.BlockSpec((BN, W16), lambda i: (i, 0)),
        pl.BlockSpec((NC, BN, W16), lambda i: (0, i, 0)),
    ],
    out_specs=pl.BlockSpec((BN, W16), lambda i: (i, 0)),
    out_shape=jax.ShapeDtypeStruct((NPAD, W16), jnp.float32),
)


def kernel(x, edge_index, W1, b1, W2, b2, Ws, bs):
    n = x.shape[0]
    e = edge_index.shape[1]
    xp = jnp.pad(x, ((0, NPAD - n), (0, 0)))
    srcp = jnp.pad(edge_index[0], (0, EPAD - e),
                   constant_values=NPAD - 1).reshape(NCHT, K)
    dstp = jnp.pad(edge_index[1], (0, EPAD - e),
                   constant_values=NPAD - 1).reshape(NCHT, K)
    w2p = jnp.pad(W2, ((0, 0), (0, W16 - W2.shape[1])))
    wsp = jnp.pad(Ws, ((0, 0), (0, W16 - Ws.shape[1])))
    br = jnp.pad(bs + b2, (0, W16 - bs.shape[0])).reshape(1, W16)
    b1r = b1.reshape(1, D)

    degp = _deg(dstp)
    hs = _tcb(xp, W1, degp)
    p = _agg128(hs, srcp, dstp)
    h2s, base = _tcd(p, hs, degp, b1r, w2p, xp, wsp, br)
    q = _agg16(h2s, srcp, dstp)
    outp = _tcf(q, h2s, base, degp)
    return outp[:n, :2]


# speedup vs baseline: 1.0648x; 1.0648x over previous
import functools

import jax
import jax.numpy as jnp
from jax import lax
from jax.experimental import pallas as pl
from jax.experimental.pallas import tpu as pltpu
from jax.experimental.pallas import tpu_sc as plsc

D = 128
W16 = 16

NPAD = 10240
EPAD = 327680
NC = 2
NS = 16
NW = NC * NS
K = 128
NCHT = EPAD // K
CTOT = NCHT // NS
RPS = NPAD // NS

_MESH = dict(core_axis_name="c", subcore_axis_name="s", num_cores=NC,
             num_subcores=NS)


def _make_agg(d, sup, c0):
    mesh = plsc.VectorSubcoreMesh(**_MESH)
    c1 = CTOT - c0
    assert c0 % sup == 0 and c1 % sup == 0

    @functools.partial(
        pl.kernel,
        out_type=jax.ShapeDtypeStruct((NC, NPAD, d), jnp.float32),
        mesh=mesh,
        compiler_params=pltpu.CompilerParams(use_tc_tiling_on_sc=False),
        scratch_types=[
            pltpu.VMEM((sup, K), jnp.int32),
            pltpu.VMEM((sup, K), jnp.int32),
            pltpu.VMEM((K, d), jnp.float32),
            pltpu.VMEM((K, d), jnp.float32),
            pltpu.VMEM_SHARED((NPAD, d), jnp.float32),
            pltpu.SemaphoreType.DMA,
            pltpu.SemaphoreType.DMA,
        ],
    )
    def agg(tbl, srcp, dstp, out, src_a, dst_a, rows0, rows1, acc, sem0,
            sem1):
        c = lax.axis_index("c")
        s = lax.axis_index("s")
        cbase = jnp.where(c == 0, s * c0, NS * c0 + s * c1)
        nsup = jnp.where(c == 0, c0 // sup, c1 // sup)

        def zrow(r, _):
            def zcol(cc, _):
                rows0[r, pl.ds(cc * 16, 16)] = jnp.zeros((16,), jnp.float32)
                return 0
            return lax.fori_loop(0, d // 16, zcol, 0)
        lax.fori_loop(0, K, zrow, 0)

        def zstripe(j, _):
            pltpu.sync_copy(rows0, acc.at[pl.ds(s * RPS + j * K, K)])
            return 0
        lax.fori_loop(0, RPS // K, zstripe, 0)
        plsc.subcore_barrier()

        def start_g(i, buf, sem):
            pltpu.async_copy(tbl.at[src_a.at[i]], buf, sem)

        def wait_g(buf, sem):
            pltpu.make_async_copy(tbl.at[pl.ds(0, K)], buf, sem).wait()

        def scat(i, buf):
            pltpu.sync_copy(buf, acc.at[dst_a.at[i]], add=True)

        def super_chunk(g, _):
            sbase = cbase + g * sup
            pltpu.sync_copy(srcp.at[pl.ds(sbase, sup)], src_a)
            pltpu.sync_copy(dstp.at[pl.ds(sbase, sup)], dst_a)
            start_g(0, rows0, sem0)

            def loop(j, _):
                i0 = 2 * j
                start_g(i0 + 1, rows1, sem1)
                wait_g(rows0, sem0)
                scat(i0, rows0)
                start_g(i0 + 2, rows0, sem0)
                wait_g(rows1, sem1)
                scat(i0 + 1, rows1)
                return 0
            lax.fori_loop(0, sup // 2 - 1, loop, 0)

            start_g(sup - 1, rows1, sem1)
            wait_g(rows0, sem0)
            scat(sup - 2, rows0)
            wait_g(rows1, sem1)
            scat(sup - 1, rows1)
            return 0
        lax.fori_loop(0, nsup, super_chunk, 0)

        plsc.subcore_barrier()
        pltpu.sync_copy(acc.at[pl.ds(s * RPS, RPS)],
                        out.at[c, pl.ds(s * RPS, RPS)])

    return agg


_agg128 = _make_agg(D, 8, 120)
_agg16 = _make_agg(W16, 16, 96)


def _make_deg():
    mesh = plsc.VectorSubcoreMesh(**_MESH)
    npw = NCHT // NW

    @functools.partial(
        pl.kernel,
        out_type=jax.ShapeDtypeStruct((NC, NPAD, W16), jnp.float32),
        mesh=mesh,
        compiler_params=pltpu.CompilerParams(use_tc_tiling_on_sc=False),
        scratch_types=[
            pltpu.VMEM((npw, K), jnp.int32),
            pltpu.VMEM((K, W16), jnp.float32),
            pltpu.VMEM((K, W16), jnp.float32),
            pltpu.VMEM_SHARED((NPAD, W16), jnp.float32),
            pltpu.SemaphoreType.DMA,
        ],
    )
    def deg(dstp, out, dst_a, ones_v, zb, acc, sem):
        c = lax.axis_index("c")
        s = lax.axis_index("s")
        wid = s * NC + c

        pltpu.sync_copy(dstp.at[pl.ds(wid * npw, npw)], dst_a)

        def fill(r, _):
            ones_v[r, pl.ds(0, 16)] = jnp.ones((16,), jnp.float32)
            zb[r, pl.ds(0, 16)] = jnp.zeros((16,), jnp.float32)
            return 0
        lax.fori_loop(0, K, fill, 0)

        def zstripe(j, _):
            pltpu.sync_copy(zb, acc.at[pl.ds(s * RPS + j * K, K)])
            return 0
        lax.fori_loop(0, RPS // K, zstripe, 0)
        plsc.subcore_barrier()

        def chunk(i, _):
            pltpu.sync_copy(ones_v, acc.at[dst_a.at[i]], add=True)
            return 0
        lax.fori_loop(0, npw, chunk, 0)
        plsc.subcore_barrier()

        pltpu.sync_copy(acc.at[pl.ds(s * RPS, RPS)],
                        out.at[c, pl.ds(s * RPS, RPS)])

    return deg


_deg = _make_deg()

BN = 256
GRID = NPAD // BN


def _dinv_of(degr):
    deg = (jnp.sum(degr[0], axis=1, keepdims=True)
           + jnp.sum(degr[1], axis=1, keepdims=True)) * (1.0 / W16) + 1.0
    return lax.rsqrt(deg)


def _tcb_body(xr, w1r, degr, hsr):
    dinv = _dinv_of(degr)
    hsr[...] = jnp.dot(xr[...], w1r[...],
                       preferred_element_type=jnp.float32) * dinv


_tcb = pl.pallas_call(
    _tcb_body,
    grid=(GRID,),
    in_specs=[
        pl.BlockSpec((BN, D), lambda i: (i, 0)),
        pl.BlockSpec((D, D), lambda i: (0, 0)),
        pl.BlockSpec((NC, BN, W16), lambda i: (0, i, 0)),
    ],
    out_specs=pl.BlockSpec((BN, D), lambda i: (i, 0)),
    out_shape=jax.ShapeDtypeStruct((NPAD, D), jnp.float32),
)


def _tcd_body(pr, hsr, degr, b1r, w2r, xr, wsr, br, h2sr, baser):
    dinv = _dinv_of(degr)
    h = jnp.maximum((pr[0] + pr[1] + hsr[...]) * dinv + b1r[...], 0.0)
    h2sr[...] = jnp.dot(h, w2r[...], preferred_element_type=jnp.float32) * dinv
    baser[...] = jnp.dot(xr[...], wsr[...],
                         preferred_element_type=jnp.float32) + br[...]


_tcd = pl.pallas_call(
    _tcd_body,
    grid=(GRID,),
    in_specs=[
        pl.BlockSpec((NC, BN, D), lambda i: (0, i, 0)),
        pl.BlockSpec((BN, D), lambda i: (i, 0)),
        pl.BlockSpec((NC, BN, W16), lambda i: (0, i, 0)),
        pl.BlockSpec((1, D), lambda i: (0, 0)),
        pl.BlockSpec((D, W16), lambda i: (0, 0)),
        pl.BlockSpec((BN, D), lambda i: (i, 0)),
        pl.BlockSpec((D, W16), lambda i: (0, 0)),
        pl.BlockSpec((1, W16), lambda i: (0, 0)),
    ],
    out_specs=[
        pl.BlockSpec((BN, W16), lambda i: (i, 0)),
        pl.BlockSpec((BN, W16), lambda i: (i, 0)),
    ],
    out_shape=[
        jax.ShapeDtypeStruct((NPAD, W16), jnp.float32),
        jax.ShapeDtypeStruct((NPAD, W16), jnp.float32),
    ],
)


def _tcf_body(qr, h2sr, baser, degr, outr):
    dinv = _dinv_of(degr)
    outr[...] = (qr[0] + qr[1] + h2sr[...]) * dinv + baser[...]


_tcf = pl.pallas_call(
    _tcf_body,
    grid=(GRID,),
    in_specs=[
        pl.BlockSpec((NC, BN, W16), lambda i: (0, i, 0)),
        pl.BlockSpec((BN, W16), lambda i: (i, 0)),
        pl.BlockSpec((BN, W16), lambda i: (i, 0)),
        pl.BlockSpec((NC, BN, W16), lambda i: (0, i, 0)),
    ],
    out_specs=pl.BlockSpec((BN, W16), lambda i: (i, 0)),
    out_shape=jax.ShapeDtypeStruct((NPAD, W16), jnp.float32),
)


def kernel(x, edge_index, W1, b1, W2, b2, Ws, bs):
    n = x.shape[0]
    e = edge_index.shape[1]
    xp = jnp.pad(x, ((0, NPAD - n), (0, 0)))
    srcp = jnp.pad(edge_index[0], (0, EPAD - e),
                   constant_values=NPAD - 1).reshape(NCHT, K)
    dstp = jnp.pad(edge_index[1], (0, EPAD - e),
                   constant_values=NPAD - 1).reshape(NCHT, K)
    w2p = jnp.pad(W2, ((0, 0), (0, W16 - W2.shape[1])))
    wsp = jnp.pad(Ws, ((0, 0), (0, W16 - Ws.shape[1])))
    br = jnp.pad(bs + b2, (0, W16 - bs.shape[0])).reshape(1, W16)
    b1r = b1.reshape(1, D)

    degp = _deg(dstp)
    hs = _tcb(xp, W1, degp)
    p = _agg128(hs, srcp, dstp)
    h2s, base = _tcd(p, hs, degp, b1r, w2p, xp, wsp, br)
    q = _agg16(h2s, srcp, dstp)
    outp = _tcf(q, h2s, base, degp)
    return outp[:n, :2]
